# baseline (speedup 1.0000x reference)
import functools

import jax
import jax.numpy as jnp
from jax import lax
from jax.experimental import pallas as pl
from jax.experimental.pallas import tpu as pltpu
from jax.experimental.pallas import tpu_sc as plsc

BATCH = 16384
EMBED_K = 64
LANES = 16

_info = plsc.get_sparse_core_info()
NC, NS = _info.num_cores, _info.num_subcores
NW = NC * NS
B_PER_W = BATCH // NW
A_ROWS = 320
B_ROWS = B_PER_W - A_ROWS
WINDOW = 64

_mesh = plsc.VectorSubcoreMesh(core_axis_name="c", subcore_axis_name="s")


@functools.partial(
    pl.kernel,
    out_type=(
        jax.ShapeDtypeStruct((BATCH,), jnp.float32),
        jax.ShapeDtypeStruct((BATCH, EMBED_K), jnp.float32),
        jax.ShapeDtypeStruct((BATCH, EMBED_K), jnp.float32),
    ),
    mesh=_mesh,
    compiler_params=pltpu.CompilerParams(needs_layout_passes=False),
    scratch_types=[
        pltpu.VMEM((B_PER_W,), jnp.int32),
        pltpu.VMEM((B_PER_W,), jnp.int32),
        pltpu.VMEM((A_ROWS, EMBED_K), jnp.float32),
        pltpu.VMEM((A_ROWS, EMBED_K), jnp.float32),
        pltpu.VMEM((B_PER_W,), jnp.float32),
        pltpu.SemaphoreType.DMA,
        pltpu.SemaphoreType.DMA,
        pltpu.SemaphoreType.DMA,
        pltpu.SemaphoreType.DMA,
    ],
)
def _bpr_kernel(users_hbm, items_hbm, gu_hbm, gi_hbm,
                xui_hbm, gu_out_hbm, gi_out_hbm,
                idx_u, idx_i, rows_u, rows_i, xui_v,
                sem_u, sem_i, sem_du, sem_di):
    wid = lax.axis_index("s") * NC + lax.axis_index("c")
    base = wid * B_PER_W

    pltpu.sync_copy(users_hbm.at[pl.ds(base, B_PER_W)], idx_u)
    pltpu.sync_copy(items_hbm.at[pl.ds(base, B_PER_W)], idx_i)

    def direct_group(g, _):
        gb = A_ROWS + g * LANES
        ob = base + gb
        vu = idx_u[pl.ds(gb, LANES)]
        vi = idx_i[pl.ds(gb, LANES)]
        for rr in range(LANES):
            pltpu.async_copy(gu_hbm.at[vu[rr]], gu_out_hbm.at[ob + rr], sem_du)
            pltpu.async_copy(gi_hbm.at[vi[rr]], gi_out_hbm.at[ob + rr], sem_di)
        return 0

    lax.fori_loop(0, B_ROWS // LANES, direct_group, 0)

    def drain_one(sem):
        pltpu.make_async_copy(gu_hbm.at[0], rows_u.at[0], sem).wait()

    gwin = WINDOW // LANES

    def fetch_group(g, _):
        gb = g * LANES
        vu = idx_u[pl.ds(gb, LANES)]
        vi = idx_i[pl.ds(gb, LANES)]
        for rr in range(LANES):
            pltpu.async_copy(gu_hbm.at[vu[rr]], rows_u.at[gb + rr], sem_u)
            pltpu.async_copy(gi_hbm.at[vi[rr]], rows_i.at[gb + rr], sem_i)

        @pl.when(g >= gwin)
        def _():
            for _ in range(LANES):
                drain_one(sem_u)
                drain_one(sem_i)

        return 0

    lax.fori_loop(0, A_ROWS // LANES, fetch_group, 0)
    for _ in range(WINDOW):
        drain_one(sem_u)
        drain_one(sem_i)

    pltpu.sync_copy(rows_u, gu_out_hbm.at[pl.ds(base, A_ROWS)])
    pltpu.sync_copy(rows_i, gi_out_hbm.at[pl.ds(base, A_ROWS)])

    lane_iota = jnp.arange(LANES, dtype=jnp.int32)

    def make_dots(pbase, nrows):
        def group_body(g, _):
            rbase = g * LANES
            acc = jnp.zeros((LANES,), jnp.float32)
            for rr in range(LANES):
                r = rbase + rr
                s = jnp.zeros((LANES,), jnp.float32)
                for c in range(EMBED_K // LANES):
                    u = rows_u[r, pl.ds(c * LANES, LANES)]
                    v = rows_i[r, pl.ds(c * LANES, LANES)]
                    s = s + u * v
                acc = jnp.where(lane_iota == rr, jnp.sum(s), acc)
            xui_v[pl.ds(pbase + rbase, LANES)] = acc
            return 0

        lax.fori_loop(0, nrows // LANES, group_body, 0)

    make_dots(0, A_ROWS)

    pltpu.make_async_copy(gu_hbm.at[pl.ds(0, B_ROWS)],
                          gu_out_hbm.at[pl.ds(base + A_ROWS, B_ROWS)],
                          sem_du).wait()
    pltpu.make_async_copy(gi_hbm.at[pl.ds(0, B_ROWS)],
                          gi_out_hbm.at[pl.ds(base + A_ROWS, B_ROWS)],
                          sem_di).wait()
    pltpu.sync_copy(gu_out_hbm.at[pl.ds(base + A_ROWS, B_ROWS)],
                    rows_u.at[pl.ds(0, B_ROWS)])
    pltpu.sync_copy(gi_out_hbm.at[pl.ds(base + A_ROWS, B_ROWS)],
                    rows_i.at[pl.ds(0, B_ROWS)])
    make_dots(A_ROWS, B_ROWS)

    pltpu.sync_copy(xui_v, xui_hbm.at[pl.ds(base, B_PER_W)])


def kernel(users, items, Gu, Gi):
    return _bpr_kernel(users, items, Gu, Gi)
